# baseline (speedup 1.0000x reference)
import functools

import jax
import jax.numpy as jnp
from jax import lax
from jax.experimental import pallas as pl
from jax.experimental.pallas import tpu as pltpu
from jax.experimental.pallas import tpu_sc as plsc

D = 64
CHUNK = 128
NWORKERS = 32
LANES = 16


def _sc_gather_add(table, idx1, idx2):
    rows = idx1.shape[0]
    rows_per_w = rows // NWORKERS
    nch = rows_per_w // CHUNK
    mesh = plsc.VectorSubcoreMesh(core_axis_name="c", subcore_axis_name="s")

    @functools.partial(
        pl.kernel,
        mesh=mesh,
        out_type=jax.ShapeDtypeStruct((rows, D), jnp.float32),
        scratch_types=[
            pltpu.VMEM((rows_per_w,), jnp.int32),
            pltpu.VMEM((rows_per_w,), jnp.int32),
            pltpu.VMEM((CHUNK, D), jnp.float32),
            pltpu.VMEM((CHUNK, D), jnp.float32),
            pltpu.SemaphoreType.DMA,
            pltpu.SemaphoreType.DMA,
        ],
        compiler_params=pltpu.CompilerParams(use_tc_tiling_on_sc=False),
    )
    def k(enc_hbm, i1_hbm, i2_hbm, out_hbm, i1_v, i2_v, r1_v, r2_v, sem1, sem2):
        wid = lax.axis_index("s") * 2 + lax.axis_index("c")
        rbase = wid * rows_per_w
        pltpu.sync_copy(i1_hbm.at[pl.ds(rbase, rows_per_w)], i1_v)
        pltpu.sync_copy(i2_hbm.at[pl.ds(rbase, rows_per_w)], i2_v)

        def chunk_body(j, _):
            isl = pl.ds(j * CHUNK, CHUNK)
            cp1 = pltpu.async_copy(enc_hbm.at[i1_v.at[isl]], r1_v, sem1)
            cp2 = pltpu.async_copy(enc_hbm.at[i2_v.at[isl]], r2_v, sem2)
            cp1.wait()
            cp2.wait()

            def add_body(i, _):
                for c in range(D // LANES):
                    sl = pl.ds(c * LANES, LANES)
                    r1_v[i, sl] = r1_v[i, sl] + r2_v[i, sl]
                return 0

            lax.fori_loop(0, CHUNK, add_body, 0)
            row_start = rbase + j * CHUNK
            pltpu.sync_copy(r1_v, out_hbm.at[pl.ds(row_start, CHUNK)])
            return 0

        lax.fori_loop(0, nch, chunk_body, 0)

    return k(table, idx1, idx2)


def _tc2_body(latp_ref, pe_ref, w1_ref, w2_ref, b_ref, out_ref):
    m = lax.dot_general(w2_ref[...], latp_ref[0], (((1,), (0,)), ((), ())),
                        preferred_element_type=jnp.float32)
    pe_t = lax.dot_general(w1_ref[...], pe_ref[0], (((1,), (1,)), ((), ())),
                           preferred_element_type=jnp.float32)
    out_ref[0] = m + pe_t + b_ref[...]


def _tc2(latp, pe3, w1, w2, bcol):
    npart, dm, nb = latp.shape
    return pl.pallas_call(
        _tc2_body,
        grid=(npart,),
        in_specs=[
            pl.BlockSpec((1, dm, nb), lambda p: (p, 0, 0)),
            pl.BlockSpec((1, nb, dm), lambda p: (p, 0, 0)),
            pl.BlockSpec((dm, dm), lambda p: (0, 0)),
            pl.BlockSpec((dm, dm), lambda p: (0, 0)),
            pl.BlockSpec((dm, 1), lambda p: (0, 0)),
        ],
        compiler_params=pltpu.CompilerParams(
            dimension_semantics=("arbitrary",),
        ),
        out_specs=pl.BlockSpec((1, dm, nb), lambda p: (p, 0, 0)),
        out_shape=jax.ShapeDtypeStruct((npart, dm, nb), jnp.float32),
    )(latp, pe3, w1, w2, bcol)


def kernel(dfn, dfn_fa, tokenized_parts_latent, encoding, W, b):
    nb, npart, dm = tokenized_parts_latent.shape
    idx1 = dfn.T.reshape(-1).astype(jnp.int32)
    idx2 = dfn_fa.T.reshape(-1).astype(jnp.int32)
    pe = _sc_gather_add(encoding, idx1, idx2)
    pe3 = pe.reshape(npart, nb, dm)

    latp = jnp.transpose(tokenized_parts_latent, (1, 2, 0))
    outp = _tc2(latp, pe3, W[:, :dm], W[:, dm:], b.reshape(dm, 1))
    return jnp.transpose(outp, (2, 0, 1))
